# Initial kernel scaffold; baseline (speedup 1.0000x reference)
import jax
import jax.numpy as jnp
from jax.experimental import pallas as pl


def kernel(table, batch):
    raise NotImplementedError("write your pallas kernel here")


# speedup vs baseline: 2.6808x; 2.6808x over previous
import functools

import jax
import jax.numpy as jnp
from jax import lax
from jax.experimental import pallas as pl
from jax.experimental.pallas import tpu as pltpu
from jax.experimental.pallas import tpu_sc as plsc

VOCAB = 1000000
DIM = 32
B = 16384
L = 50

NC = 2
NS = 16
NW = NC * NS
SPW = B // NW
C = 16
ROWS = C * L
NCHUNK = SPW // C
IW = 100
NG = ROWS // IW
IDX_ROWS_TOTAL = B * L // IW

_mesh = plsc.VectorSubcoreMesh(core_axis_name="c", subcore_axis_name="s")


@functools.partial(
    pl.kernel,
    out_type=jax.ShapeDtypeStruct((B, DIM), jnp.float32),
    mesh=_mesh,
    scratch_types=[
        pltpu.VMEM((NG, IW), jnp.int32),
        pltpu.VMEM((ROWS, DIM), jnp.float32),
        pltpu.VMEM((C, DIM), jnp.float32),
        pltpu.SemaphoreType.DMA,
    ],
    compiler_params=pltpu.CompilerParams(use_tc_tiling_on_sc=False),
)
def _embed_kernel(table_hbm, batch_hbm, out_hbm, idx_v, rows_v, out_v, sem):
    wid = lax.axis_index("s") * NC + lax.axis_index("c")

    def chunk_body(ci, _):
        idx_row0 = wid * (SPW * L // IW) + ci * NG
        pltpu.sync_copy(batch_hbm.at[pl.ds(idx_row0, NG)], idx_v)
        copies = [
            pltpu.async_copy(
                table_hbm.at[idx_v.at[j]],
                rows_v.at[pl.ds(j * IW, IW)],
                sem,
            )
            for j in range(NG)
        ]
        for cp in copies:
            cp.wait()

        def acc_body(l, accs):
            out = []
            for s in range(C):
                a0, a1 = accs[s]
                r = s * L + l
                out.append((a0 + rows_v[r, 0:16], a1 + rows_v[r, 16:32]))
            return tuple(out)

        zero = jnp.zeros((16,), jnp.float32)
        init = tuple((zero, zero) for _ in range(C))
        accs = lax.fori_loop(0, L, acc_body, init)
        scale = jnp.float32(1.0 / L)
        for s in range(C):
            a0, a1 = accs[s]
            out_v[s, 0:16] = a0 * scale
            out_v[s, 16:32] = a1 * scale

        base = wid * SPW + ci * C
        pltpu.sync_copy(out_v, out_hbm.at[pl.ds(base, C)])
        return 0

    lax.fori_loop(0, NCHUNK, chunk_body, 0)


def kernel(table, batch):
    batch_r = batch.reshape(IDX_ROWS_TOTAL, IW)
    return _embed_kernel(table, batch_r)
